# baseline (speedup 1.0000x reference)
import functools

import jax
import jax.numpy as jnp
from jax.experimental import pallas as pl
from jax.experimental.pallas import tpu as pltpu

KERNEL = 25
TOPK = 2


def _fused_kernel(x_ref, gs_ref, gt_ref, ws0_ref, ws1_ref, wt0_ref, wt1_ref,
                  bts_ref, btt_ref, o_ref, pm_ref, a_scr, c_scr,
                  *, B, V, L, E, P):
    m = pl.program_id(0)
    e = pl.program_id(1)
    ph = pl.program_id(2)
    P2 = P // 2
    first = (m == 0) & (e == 0)

    @pl.when(first & (ph == 0))
    def _prologue():
        pad = (KERNEL - 1) // 2
        pm_sum = jnp.zeros((E, V), jnp.float32)
        for b in range(B):
            xb = x_ref[b]
            a3 = xb[0:L - 16] + xb[8:L - 8] + xb[16:L]
            sum8 = a3[0:L - 24]
            for c in range(1, 8):
                sum8 = sum8 + a3[c:c + L - 24]
            core = sum8 + xb[24:L]
            front_rows = []
            p = xb[0:1, :]
            for j in range(1, 24):
                p = p + xb[j:j + 1, :]
                if j >= pad:
                    front_rows.append(p + float(24 - j) * xb[0:1, :])
            back_rows = []
            s = xb[L - 1:L, :]
            for j in range(L - 2, L - 25, -1):
                s = s + xb[j:j + 1, :]
                if j <= L - 1 - pad:
                    back_rows.append(s + float(j - L + 25) * xb[L - 1:L, :])
            back_rows.reverse()
            mov = jnp.concatenate(front_rows + [core] + back_rows,
                                  axis=0) * (1.0 / KERNEL)
            sea = xb - mov
            sl = slice(b * V, (b + 1) * V)
            a_scr[0, :, sl] = sea
            a_scr[1, :, sl] = mov
            for mi, (a, g_ref) in enumerate(((sea, gs_ref), (mov, gt_ref))):
                logits = jax.lax.dot_general(
                    g_ref[...], a, (((1,), (0,)), ((), ())),
                    preferred_element_type=jnp.float32)
                z = logits - jnp.max(logits, axis=0, keepdims=True)
                ez = jnp.exp(z)
                probs = ez / jnp.sum(ez, axis=0, keepdims=True)
                iota = jax.lax.broadcasted_iota(jnp.int32, (E, V), 0)
                m1 = jnp.max(probs, axis=0, keepdims=True)
                sel1 = jnp.min(jnp.where(probs == m1, iota, E), axis=0,
                               keepdims=True)
                oh1 = iota == sel1
                masked = jnp.where(oh1, -1.0, probs)
                m2 = jnp.max(masked, axis=0, keepdims=True)
                sel2 = jnp.min(jnp.where(masked == m2, iota, E), axis=0,
                               keepdims=True)
                oh2 = iota == sel2
                c_scr[mi, :, sl] = jnp.where(oh1 | oh2, probs, 0.0)
                if mi == 1:
                    pm_sum = pm_sum + probs * (1.0 / B)
        pm_ref[...] = pm_sum

    P4 = P // 4

    def accumulate(mi, w_ref, bt_ref, q):
        y = jax.lax.dot_general(
            w_ref[0], a_scr[mi], (((1,), (0,)), ((), ())),
            preferred_element_type=jnp.float32)
        cm = c_scr[mi]
        sub = jax.lax.broadcasted_iota(jnp.int32, cm.shape, 0)
        crow = jnp.sum(jnp.where(sub == e, cm, 0.0), axis=0, keepdims=True)
        prow = pl.multiple_of((2 * ph + q) * P4, P4)
        bt = bt_ref[pl.ds(prow, P4), :]
        lane = jax.lax.broadcasted_iota(jnp.int32, bt.shape, 1)
        bcol = jnp.sum(jnp.where(lane == e, bt, 0.0), axis=1, keepdims=True)
        contrib = crow * (y + bcol)
        for b in range(B):
            blk = contrib[:, b * V:(b + 1) * V]

            @pl.when(first)
            def _():
                o_ref[b, pl.ds(prow, P4), :] = blk

            @pl.when(~first)
            def _():
                o_ref[b, pl.ds(prow, P4), :] += blk

    @pl.when(m == 0)
    def _():
        accumulate(0, ws0_ref, bts_ref, 0)
        accumulate(0, ws1_ref, bts_ref, 1)

    @pl.when(m == 1)
    def _():
        accumulate(1, wt0_ref, btt_ref, 0)
        accumulate(1, wt1_ref, btt_ref, 1)


def kernel(x, Gw_sea, Ew_sea, Eb_sea, Gw_trend, Ew_trend, Eb_trend):
    B, L, V = x.shape
    E, P, _ = Ew_sea.shape
    T = B * V

    out, pm = pl.pallas_call(
        functools.partial(_fused_kernel, B=B, V=V, L=L, E=E, P=P),
        grid=(2, E, 2),
        in_specs=[
            pl.BlockSpec((B, L, V), lambda m, e, ph: (0, 0, 0)),
            pl.BlockSpec((E, L), lambda m, e, ph: (0, 0)),
            pl.BlockSpec((E, L), lambda m, e, ph: (0, 0)),
            pl.BlockSpec((1, P // 4, L),
                         lambda m, e, ph: ((1 - m) * e + m * (E - 1),
                                           (1 - m) * 2 * ph + m * 2, 0)),
            pl.BlockSpec((1, P // 4, L),
                         lambda m, e, ph: ((1 - m) * e + m * (E - 1),
                                           (1 - m) * (2 * ph + 1) + m * 3, 0)),
            pl.BlockSpec((1, P // 4, L),
                         lambda m, e, ph: (m * e, m * 2 * ph, 0)),
            pl.BlockSpec((1, P // 4, L),
                         lambda m, e, ph: (m * e, m * (2 * ph + 1), 0)),
            pl.BlockSpec((P, E), lambda m, e, ph: (0, 0)),
            pl.BlockSpec((P, E), lambda m, e, ph: (0, 0)),
        ],
        out_specs=[
            pl.BlockSpec((B, P, V), lambda m, e, ph: (0, 0, 0)),
            pl.BlockSpec((E, V), lambda m, e, ph: (0, 0)),
        ],
        out_shape=[
            jax.ShapeDtypeStruct((B, P, V), jnp.float32),
            jax.ShapeDtypeStruct((E, V), jnp.float32),
        ],
        scratch_shapes=[
            pltpu.VMEM((2, L, T), jnp.float32),
            pltpu.VMEM((2, E, T), jnp.float32),
        ],
    )(x, Gw_sea, Gw_trend, Ew_sea, Ew_sea, Ew_trend, Ew_trend,
      Eb_sea.T, Eb_trend.T)

    return out, pm.T


# speedup vs baseline: 1.0848x; 1.0848x over previous
import functools

import jax
import jax.numpy as jnp
from jax.experimental import pallas as pl
from jax.experimental.pallas import tpu as pltpu

KERNEL = 25
TOPK = 2


def _fused_kernel(x_ref, gs_ref, gt_ref, ws_ref, wt_ref, bts_ref, btt_ref,
                  o_ref, pm_ref, a_scr, c_scr, *, B, V, L, E, P):
    m = pl.program_id(0)
    e = pl.program_id(1)
    ph = pl.program_id(2)
    P2 = P // 2
    first = (m == 0) & (e == 0)

    @pl.when(first & (ph == 0))
    def _prologue():
        pad = (KERNEL - 1) // 2
        pm_sum = jnp.zeros((E, V), jnp.float32)
        for b in range(B):
            xb = x_ref[b]
            a3 = xb[0:L - 16] + xb[8:L - 8] + xb[16:L]
            sum8 = a3[0:L - 24]
            for c in range(1, 8):
                sum8 = sum8 + a3[c:c + L - 24]
            core = sum8 + xb[24:L]
            front_rows = []
            p = xb[0:1, :]
            for j in range(1, 24):
                p = p + xb[j:j + 1, :]
                if j >= pad:
                    front_rows.append(p + float(24 - j) * xb[0:1, :])
            back_rows = []
            s = xb[L - 1:L, :]
            for j in range(L - 2, L - 25, -1):
                s = s + xb[j:j + 1, :]
                if j <= L - 1 - pad:
                    back_rows.append(s + float(j - L + 25) * xb[L - 1:L, :])
            back_rows.reverse()
            mov = jnp.concatenate(front_rows + [core] + back_rows,
                                  axis=0) * (1.0 / KERNEL)
            sea = xb - mov
            sl = slice(b * V, (b + 1) * V)
            a_scr[0, :, sl] = sea
            a_scr[1, :, sl] = mov
            for mi, (a, g_ref) in enumerate(((sea, gs_ref), (mov, gt_ref))):
                logits = jax.lax.dot_general(
                    g_ref[...], a, (((1,), (0,)), ((), ())),
                    preferred_element_type=jnp.float32)
                z = logits - jnp.max(logits, axis=0, keepdims=True)
                ez = jnp.exp(z)
                probs = ez / jnp.sum(ez, axis=0, keepdims=True)
                iota = jax.lax.broadcasted_iota(jnp.int32, (E, V), 0)
                m1 = jnp.max(probs, axis=0, keepdims=True)
                sel1 = jnp.min(jnp.where(probs == m1, iota, E), axis=0,
                               keepdims=True)
                oh1 = iota == sel1
                masked = jnp.where(oh1, -1.0, probs)
                m2 = jnp.max(masked, axis=0, keepdims=True)
                sel2 = jnp.min(jnp.where(masked == m2, iota, E), axis=0,
                               keepdims=True)
                oh2 = iota == sel2
                c_scr[mi, :, sl] = jnp.where(oh1 | oh2, probs, 0.0)
                if mi == 1:
                    pm_sum = pm_sum + probs * (1.0 / B)
        pm_ref[...] = pm_sum

    def accumulate(mi, w_ref, bt_ref):
        y = jax.lax.dot_general(
            w_ref[0], a_scr[mi], (((1,), (0,)), ((), ())),
            preferred_element_type=jnp.float32)
        cm = c_scr[mi]
        sub = jax.lax.broadcasted_iota(jnp.int32, cm.shape, 0)
        crow = jnp.sum(jnp.where(sub == e, cm, 0.0), axis=0, keepdims=True)
        prow = pl.multiple_of(ph * P2, P2)
        bt = bt_ref[pl.ds(prow, P2), :]
        lane = jax.lax.broadcasted_iota(jnp.int32, bt.shape, 1)
        bcol = jnp.sum(jnp.where(lane == e, bt, 0.0), axis=1, keepdims=True)
        contrib = crow * (y + bcol)
        for b in range(B):
            blk = contrib[:, b * V:(b + 1) * V]

            @pl.when(first)
            def _():
                o_ref[b, pl.ds(prow, P2), :] = blk

            @pl.when(~first)
            def _():
                o_ref[b, pl.ds(prow, P2), :] += blk

    @pl.when(m == 0)
    def _():
        accumulate(0, ws_ref, bts_ref)

    @pl.when(m == 1)
    def _():
        accumulate(1, wt_ref, btt_ref)


def kernel(x, Gw_sea, Ew_sea, Eb_sea, Gw_trend, Ew_trend, Eb_trend):
    B, L, V = x.shape
    E, P, _ = Ew_sea.shape
    T = B * V

    out, pm = pl.pallas_call(
        functools.partial(_fused_kernel, B=B, V=V, L=L, E=E, P=P),
        grid=(2, E, 2),
        in_specs=[
            pl.BlockSpec((B, L, V), lambda m, e, ph: (0, 0, 0)),
            pl.BlockSpec((E, L), lambda m, e, ph: (0, 0)),
            pl.BlockSpec((E, L), lambda m, e, ph: (0, 0)),
            pl.BlockSpec((1, P // 2, L),
                         lambda m, e, ph: ((1 - m) * e + m * (E - 1),
                                           (1 - m) * ph + m, 0)),
            pl.BlockSpec((1, P // 2, L),
                         lambda m, e, ph: (m * e, m * ph, 0)),
            pl.BlockSpec((P, E), lambda m, e, ph: (0, 0)),
            pl.BlockSpec((P, E), lambda m, e, ph: (0, 0)),
        ],
        out_specs=[
            pl.BlockSpec((B, P, V), lambda m, e, ph: (0, 0, 0)),
            pl.BlockSpec((E, V), lambda m, e, ph: (0, 0)),
        ],
        out_shape=[
            jax.ShapeDtypeStruct((B, P, V), jnp.float32),
            jax.ShapeDtypeStruct((E, V), jnp.float32),
        ],
        scratch_shapes=[
            pltpu.VMEM((2, L, T), jnp.float32),
            pltpu.VMEM((2, E, T), jnp.float32),
        ],
    )(x, Gw_sea, Gw_trend, Ew_sea, Ew_trend, Eb_sea.T, Eb_trend.T)

    return out, pm.T
